# Initial kernel scaffold; baseline (speedup 1.0000x reference)
import functools

import jax
import jax.numpy as jnp
from jax import lax
from jax.experimental import pallas as pl
from jax.experimental.pallas import tpu as pltpu
from jax.experimental.pallas import tpu_sc as plsc

N = 4096
NT = 16384
C = 128
NCH = N // C
NW = 32
QPW = NT // NW
CLIP_LO = -2.0
CLIP_HI = 2.0
_F32 = jnp.float32
_I32 = jnp.int32


def _cumsum_lanes(x, lane_n):
    n = x.shape[1]
    zero = jnp.zeros((), x.dtype)
    s = 1
    while s < n:
        sh = pltpu.roll(x, s, axis=1)
        x = x + jnp.where(lane_n >= s, sh, zero)
        s *= 2
    return x


def _fit_body(xrow_ref, xcol_ref, ycol_ref,
              ux_ref, iso_ref, nu_ref, xmin_ref, xmax_ref):
    xrow = xrow_ref[...]
    xcol = xcol_ref[...]
    ycol = ycol_ref[...]

    i_sub = lax.broadcasted_iota(_I32, (N, 1), 0)
    lane = lax.broadcasted_iota(_I32, (1, C), 1)
    lane_n = lax.broadcasted_iota(_I32, (1, N), 1)
    j128 = lax.broadcasted_iota(_I32, (C, 1), 0)
    eye = j128 == lane
    NEG = _F32(-jnp.inf)
    POS = _F32(jnp.inf)

    def t_col_to_row(colv):
        ch = []
        for c in range(NCH):
            blk = colv[c * C:(c + 1) * C, :]
            ch.append(jnp.sum(jnp.where(eye, blk, jnp.zeros((), colv.dtype)),
                              axis=0, keepdims=True))
        return jnp.concatenate(ch, axis=1)

    def t_row_to_col(rowv):
        ch = []
        for c in range(NCH):
            blk = rowv[:, c * C:(c + 1) * C]
            ch.append(jnp.sum(jnp.where(eye, blk, jnp.zeros((), rowv.dtype)),
                              axis=1, keepdims=True))
        return jnp.concatenate(ch, axis=0)

    rank_col = jnp.zeros((N, 1), _I32)
    for c in range(NCH):
        xj = xrow[:, c * C:(c + 1) * C]
        jg = lane + c * C
        hit = (xj < xcol) | ((xj == xcol) & (jg < i_sub))
        rank_col = rank_col + jnp.sum(hit.astype(_I32), axis=1, keepdims=True)

    sx_ch, sy_ch = [], []
    for c in range(NCH):
        oh = rank_col == (lane + c * C)
        sx_ch.append(jnp.max(jnp.where(oh, xcol, NEG), axis=0, keepdims=True))
        sy_ch.append(jnp.sum(jnp.where(oh, ycol, _F32(0.0)), axis=0,
                             keepdims=True))
    sx = jnp.concatenate(sx_ch, axis=1)
    sy = jnp.concatenate(sy_ch, axis=1)

    flag = (lane_n == 0) | (sx != pltpu.roll(sx, 1, axis=1))
    flag_i = flag.astype(_I32)
    seg = _cumsum_lanes(flag_i, lane_n) - 1
    n_u = jnp.sum(flag_i, axis=1, keepdims=True)

    counts = jnp.zeros((N, 1), _F32)
    sumy = jnp.zeros((N, 1), _F32)
    ux = jnp.full((N, 1), NEG)
    for c in range(NCH):
        oh = seg[:, c * C:(c + 1) * C] == i_sub
        counts = counts + jnp.sum(jnp.where(oh, _F32(1.0), _F32(0.0)),
                                  axis=1, keepdims=True)
        sumy = sumy + jnp.sum(jnp.where(oh, sy[:, c * C:(c + 1) * C],
                                        _F32(0.0)), axis=1, keepdims=True)
        ux = jnp.maximum(ux, jnp.max(jnp.where(oh, sx[:, c * C:(c + 1) * C],
                                               NEG), axis=1, keepdims=True))
    uy_col = sumy / jnp.maximum(counts, _F32(1.0))

    uy_row = t_col_to_row(uy_col)
    s_incl = _cumsum_lanes(uy_row, lane_n)
    s_excl = jnp.where(lane_n == 0, _F32(0.0),
                       pltpu.roll(s_incl, 1, axis=1))
    s_col = t_row_to_col(s_excl)

    carry = jnp.full((N, 1), POS)
    iso_ch = [None] * NCH
    for cb in reversed(range(NCH)):
        rows = (cb + 1) * C
        scol = s_col[:rows, :]
        isub = i_sub[:rows, :]
        kg = lane + cb * C
        sk = s_incl[:, cb * C:(cb + 1) * C]
        d = kg - isub + 1
        m = (sk - scol) / jnp.maximum(d, 1).astype(_F32)
        m = jnp.where((d >= 1) & (kg < n_u), m, POS)
        s = 1
        while s < C:
            sh = pltpu.roll(m, -s, axis=1)
            m = jnp.minimum(m, jnp.where(lane < C - s, sh, POS))
            s *= 2
        a = jnp.minimum(m, carry[:rows, :])
        iso_ch[cb] = jnp.max(jnp.where(isub <= kg, a, NEG),
                             axis=0, keepdims=True)
        carry = a[:, 0:1]
    iso = jnp.concatenate(iso_ch, axis=1)
    iso = jnp.clip(iso, _F32(CLIP_LO), _F32(CLIP_HI))

    ux_ref[...] = jnp.where(i_sub >= n_u, POS, ux)
    iso_ref[...] = iso
    nu_ref[...] = n_u
    xmin_ref[...] = jnp.min(xrow, axis=1, keepdims=True)
    xmax_ref[...] = jnp.max(xrow, axis=1, keepdims=True)


_fit = pl.pallas_call(
    _fit_body,
    out_shape=[
        jax.ShapeDtypeStruct((N, 1), _F32),
        jax.ShapeDtypeStruct((1, N), _F32),
        jax.ShapeDtypeStruct((1, 1), _I32),
        jax.ShapeDtypeStruct((1, 1), _F32),
        jax.ShapeDtypeStruct((1, 1), _F32),
    ],
)


def _predict(T, ux1, iso1, xmin16, xmax16, nhi16):
    mesh = plsc.VectorSubcoreMesh(core_axis_name="c", subcore_axis_name="s")

    @functools.partial(
        pl.kernel, mesh=mesh,
        out_type=jax.ShapeDtypeStruct((NT,), _F32),
        scratch_types=[
            pltpu.VMEM((N,), _F32),
            pltpu.VMEM((N,), _F32),
            pltpu.VMEM((QPW,), _F32),
            pltpu.VMEM((QPW,), _F32),
            pltpu.VMEM((16,), _F32),
            pltpu.VMEM((16,), _F32),
            pltpu.VMEM((16,), _I32),
        ],
    )
    def k(t_hbm, ux_hbm, iso_hbm, xmin_hbm, xmax_hbm, nhi_hbm, out_hbm,
          ux_v, iso_v, t_v, o_v, xmin_v, xmax_v, nhi_v):
        wid = lax.axis_index("s") * 2 + lax.axis_index("c")
        base = wid * QPW
        pltpu.sync_copy(ux_hbm, ux_v)
        pltpu.sync_copy(iso_hbm, iso_v)
        pltpu.sync_copy(t_hbm.at[pl.ds(base, QPW)], t_v)
        pltpu.sync_copy(xmin_hbm, xmin_v)
        pltpu.sync_copy(xmax_hbm, xmax_v)
        pltpu.sync_copy(nhi_hbm, nhi_v)
        xmin = xmin_v[...]
        xmax = xmax_v[...]
        nhi = nhi_v[...]

        def body(g, acc):
            t = t_v[pl.ds(g * 16, 16)]
            tc = jnp.minimum(jnp.maximum(t, xmin), xmax)
            pos = jnp.zeros((16,), _I32)
            s = N // 2
            while s >= 1:
                cand = pos + s
                probe = plsc.load_gather(ux_v, [cand - 1])
                pos = jnp.where(probe <= tc, cand, pos)
                s //= 2
            idx = jnp.clip(pos - 1, 0, nhi)
            xb = plsc.load_gather(ux_v, [idx])
            xa = plsc.load_gather(ux_v, [idx + 1])
            yb = plsc.load_gather(iso_v, [idx])
            ya = plsc.load_gather(iso_v, [idx + 1])
            slope = (ya - yb) / (xa - xb)
            o_v[pl.ds(g * 16, 16)] = yb + slope * (tc - xb)
            return acc

        lax.fori_loop(0, QPW // 16, body, 0)
        pltpu.sync_copy(o_v, out_hbm.at[pl.ds(base, QPW)])

    return k(T, ux1, iso1, xmin16, xmax16, nhi16)


def kernel(X, y, T):
    ux, iso, nu, xmin, xmax = _fit(X.reshape(1, N), X.reshape(N, 1),
                                   y.reshape(N, 1))
    nhi = jnp.maximum(nu[0, 0] - 2, 0).astype(_I32)
    return _predict(
        T, ux.reshape(N), iso.reshape(N),
        jnp.full((16,), xmin[0, 0], _F32),
        jnp.full((16,), xmax[0, 0], _F32),
        jnp.full((16,), nhi, _I32),
    )


# speedup vs baseline: 5.0114x; 5.0114x over previous
import functools

import jax
import jax.numpy as jnp
from jax import lax
from jax.experimental import pallas as pl
from jax.experimental.pallas import tpu as pltpu
from jax.experimental.pallas import tpu_sc as plsc

N = 4096
NT = 16384
C = 128
NCH = N // C
RB = 512
NRB = N // RB
NW = 32
QPW = NT // NW
CLIP_LO = -2.0
CLIP_HI = 2.0
_F32 = jnp.float32
_I32 = jnp.int32


def _fit_body(x2d_ref, y2d_ref,
              ux_ref, iso_ref, nu_ref, xmin_ref, xmax_ref,
              xcol_ref, ycol_ref, rank_ref, sx_ref, sy_ref, seg_ref,
              uyc_ref, uy2d_ref, s0_ref, sex_ref, scol_ref, carry_ref):
    lane = lax.broadcasted_iota(_I32, (1, C), 1)
    sub128 = lax.broadcasted_iota(_I32, (C, 1), 0)
    sub512 = lax.broadcasted_iota(_I32, (RB, 1), 0)
    sub32 = lax.broadcasted_iota(_I32, (NCH, 1), 0)
    lanei = lax.broadcasted_iota(_I32, (NCH, C), 1)
    subi = lax.broadcasted_iota(_I32, (NCH, C), 0)
    eye = sub128 == lane
    NEG = _F32(-jnp.inf)
    POS = _F32(jnp.inf)

    def s0(c, _):
        xr = x2d_ref[pl.ds(c, 1), :]
        yr = y2d_ref[pl.ds(c, 1), :]
        z = _F32(0.0)
        xcol_ref[pl.ds(c * C, C), :] = jnp.sum(
            jnp.where(eye, xr, z), axis=1, keepdims=True)
        ycol_ref[pl.ds(c * C, C), :] = jnp.sum(
            jnp.where(eye, yr, z), axis=1, keepdims=True)
        return 0
    lax.fori_loop(0, NCH, s0, 0, unroll=False)

    def s1(rb, _):
        xi = xcol_ref[pl.ds(rb * RB, RB), :]
        ig = rb * RB + sub512

        def inner(c, acc):
            xj = x2d_ref[pl.ds(c, 1), :]
            jg = c * C + lane
            hit = (xj < xi) | ((xj == xi) & (jg < ig))
            return acc + jnp.sum(hit.astype(_I32), axis=1, keepdims=True)

        rnk = lax.fori_loop(0, NCH, inner, jnp.zeros((RB, 1), _I32))
        rank_ref[pl.ds(rb * RB, RB), :] = rnk
        return 0
    lax.fori_loop(0, NRB, s1, 0, unroll=False)

    def s2(c, _):
        rt = c * C + lane

        def inner(rb, carry):
            mx, sm = carry
            oh = rank_ref[pl.ds(rb * RB, RB), :] == rt
            xi = xcol_ref[pl.ds(rb * RB, RB), :]
            yi = ycol_ref[pl.ds(rb * RB, RB), :]
            mx = jnp.maximum(mx, jnp.max(jnp.where(oh, xi, NEG),
                                         axis=0, keepdims=True))
            sm = sm + jnp.sum(jnp.where(oh, yi, _F32(0.0)),
                              axis=0, keepdims=True)
            return mx, sm

        mx, sm = lax.fori_loop(0, NRB, inner,
                               (jnp.full((1, C), NEG), jnp.zeros((1, C), _F32)))
        sx_ref[pl.ds(c, 1), :] = mx
        sy_ref[pl.ds(c, 1), :] = sm
        return 0
    lax.fori_loop(0, NCH, s2, 0, unroll=False)

    def prev_elem(v, fill):
        r1 = pltpu.roll(v, 1, axis=1)
        pv = jnp.where(lanei == 0, pltpu.roll(r1, 1, axis=0), r1)
        return jnp.where(subi * C + lanei == 0, fill, pv)

    def cumsum2d(v):
        s = 1
        while s < C:
            sh = pltpu.roll(v, s, axis=1)
            v = v + jnp.where(lanei >= s, sh, jnp.zeros((), v.dtype))
            s *= 2
        rt = v[:, C - 1:C]
        t = rt
        s = 1
        while s < NCH:
            sh = pltpu.roll(t, s, axis=0)
            t = t + jnp.where(sub32 >= s, sh, jnp.zeros((), v.dtype))
            s *= 2
        return v + (t - rt)

    sx2 = sx_ref[...]
    flag = (subi * C + lanei == 0) | (sx2 != prev_elem(sx2, NEG))
    flag_i = flag.astype(_I32)
    seg_ref[...] = cumsum2d(flag_i) - 1
    n_u = jnp.sum(jnp.sum(flag_i, axis=1, keepdims=True),
                  axis=0, keepdims=True)

    def s4(rb, _):
        ug = rb * RB + sub512

        def inner(c, carry):
            cnt, sm, mx = carry
            oh = seg_ref[pl.ds(c, 1), :] == ug
            syc = sy_ref[pl.ds(c, 1), :]
            sxc = sx_ref[pl.ds(c, 1), :]
            cnt = cnt + jnp.sum(jnp.where(oh, _F32(1.0), _F32(0.0)),
                                axis=1, keepdims=True)
            sm = sm + jnp.sum(jnp.where(oh, syc, _F32(0.0)),
                              axis=1, keepdims=True)
            mx = jnp.maximum(mx, jnp.max(jnp.where(oh, sxc, NEG),
                                         axis=1, keepdims=True))
            return cnt, sm, mx

        cnt, sm, mx = lax.fori_loop(
            0, NCH, inner,
            (jnp.zeros((RB, 1), _F32), jnp.zeros((RB, 1), _F32),
             jnp.full((RB, 1), NEG)))
        uyc_ref[pl.ds(rb * RB, RB), :] = sm / jnp.maximum(cnt, _F32(1.0))
        ux_ref[pl.ds(rb * RB, RB), :] = jnp.where(ug >= n_u, POS, mx)
        return 0
    lax.fori_loop(0, NRB, s4, 0, unroll=False)

    def t_uy(c, _):
        blk = uyc_ref[pl.ds(c * C, C), :]
        uy2d_ref[pl.ds(c, 1), :] = jnp.sum(
            jnp.where(eye, blk, _F32(0.0)), axis=0, keepdims=True)
        return 0
    lax.fori_loop(0, NCH, t_uy, 0, unroll=False)

    s_incl = cumsum2d(uy2d_ref[...])
    s0_ref[...] = s_incl
    sex_ref[...] = prev_elem(s_incl, _F32(0.0))

    def t_se(c, _):
        blk = sex_ref[pl.ds(c, 1), :]
        scol_ref[pl.ds(c * C, C), :] = jnp.sum(
            jnp.where(eye, blk, _F32(0.0)), axis=1, keepdims=True)
        return 0
    lax.fori_loop(0, NCH, t_se, 0, unroll=False)

    carry_ref[...] = jnp.full((N, 1), POS)

    def s5(t, _):
        cb = NCH - 1 - t
        kg = cb * C + lane
        sk = s0_ref[pl.ds(cb, 1), :]

        def inner(rb, isoacc):
            jb = rb * C + sub128
            sj = scol_ref[pl.ds(rb * C, C), :]
            d = kg - jb + 1
            m = (sk - sj) / jnp.maximum(d, 1).astype(_F32)
            m = jnp.where((d >= 1) & (kg < n_u), m, POS)
            s = 1
            while s < C:
                sh = pltpu.roll(m, C - s, axis=1)
                m = jnp.minimum(m, jnp.where(lane < C - s, sh, POS))
                s *= 2
            a = jnp.minimum(m, carry_ref[pl.ds(rb * C, C), :])
            carry_ref[pl.ds(rb * C, C), :] = a[:, 0:1]
            return jnp.maximum(isoacc,
                               jnp.max(jnp.where(jb <= kg, a, NEG),
                                       axis=0, keepdims=True))

        isoacc = lax.fori_loop(0, cb + 1, inner, jnp.full((1, C), NEG))
        iso_ref[pl.ds(cb, 1), :] = jnp.clip(isoacc, _F32(CLIP_LO),
                                            _F32(CLIP_HI))
        return 0
    lax.fori_loop(0, NCH, s5, 0, unroll=False)

    nu_ref[...] = n_u
    x2d = x2d_ref[...]
    xmin_ref[...] = jnp.min(jnp.min(x2d, axis=1, keepdims=True),
                            axis=0, keepdims=True)
    xmax_ref[...] = jnp.max(jnp.max(x2d, axis=1, keepdims=True),
                            axis=0, keepdims=True)


_FIT_OUT = [
    jax.ShapeDtypeStruct((N, 1), _F32),
    jax.ShapeDtypeStruct((NCH, C), _F32),
    jax.ShapeDtypeStruct((1, 1), _I32),
    jax.ShapeDtypeStruct((1, 1), _F32),
    jax.ShapeDtypeStruct((1, 1), _F32),
]

_FIT_SCRATCH = [
    pltpu.VMEM((N, 1), _F32),
    pltpu.VMEM((N, 1), _F32),
    pltpu.VMEM((N, 1), _I32),
    pltpu.VMEM((NCH, C), _F32),
    pltpu.VMEM((NCH, C), _F32),
    pltpu.VMEM((NCH, C), _I32),
    pltpu.VMEM((N, 1), _F32),
    pltpu.VMEM((NCH, C), _F32),
    pltpu.VMEM((NCH, C), _F32),
    pltpu.VMEM((NCH, C), _F32),
    pltpu.VMEM((N, 1), _F32),
    pltpu.VMEM((N, 1), _F32),
]

_fit = pl.pallas_call(
    _fit_body,
    out_shape=_FIT_OUT,
    scratch_shapes=_FIT_SCRATCH,
)


def _predict(T, ux1, iso1, xmin16, xmax16, nhi16):
    mesh = plsc.VectorSubcoreMesh(core_axis_name="c", subcore_axis_name="s")

    @functools.partial(
        pl.kernel, mesh=mesh,
        out_type=jax.ShapeDtypeStruct((NT,), _F32),
        compiler_params=pltpu.CompilerParams(needs_layout_passes=False),
        scratch_types=[
            pltpu.VMEM((N,), _F32),
            pltpu.VMEM((N,), _F32),
            pltpu.VMEM((QPW,), _F32),
            pltpu.VMEM((QPW,), _F32),
            pltpu.VMEM((16,), _F32),
            pltpu.VMEM((16,), _F32),
            pltpu.VMEM((16,), _I32),
        ],
    )
    def k(t_hbm, ux_hbm, iso_hbm, xmin_hbm, xmax_hbm, nhi_hbm, out_hbm,
          ux_v, iso_v, t_v, o_v, xmin_v, xmax_v, nhi_v):
        wid = lax.axis_index("s") * 2 + lax.axis_index("c")
        base = wid * QPW
        pltpu.sync_copy(ux_hbm, ux_v)
        pltpu.sync_copy(iso_hbm, iso_v)
        pltpu.sync_copy(t_hbm.at[pl.ds(base, QPW)], t_v)
        pltpu.sync_copy(xmin_hbm, xmin_v)
        pltpu.sync_copy(xmax_hbm, xmax_v)
        pltpu.sync_copy(nhi_hbm, nhi_v)
        xmin = xmin_v[...]
        xmax = xmax_v[...]
        nhi = nhi_v[...]

        def body(g, acc):
            t = t_v[pl.ds(g * 16, 16)]
            tc = jnp.minimum(jnp.maximum(t, xmin), xmax)
            pos = jnp.zeros((16,), _I32)
            s = N // 2
            while s >= 1:
                cand = pos + s
                probe = plsc.load_gather(ux_v, [cand - 1])
                pos = jnp.where(probe <= tc, cand, pos)
                s //= 2
            idx = jnp.clip(pos - 1, 0, nhi)
            xb = plsc.load_gather(ux_v, [idx])
            xa = plsc.load_gather(ux_v, [idx + 1])
            yb = plsc.load_gather(iso_v, [idx])
            ya = plsc.load_gather(iso_v, [idx + 1])
            slope = (ya - yb) / (xa - xb)
            o_v[pl.ds(g * 16, 16)] = yb + slope * (tc - xb)
            return acc

        lax.fori_loop(0, QPW // 16, body, 0)
        pltpu.sync_copy(o_v, out_hbm.at[pl.ds(base, QPW)])

    return k(T, ux1, iso1, xmin16, xmax16, nhi16)


def kernel(X, y, T):
    ux, iso, nu, xmin, xmax = _fit(X.reshape(NCH, C), y.reshape(NCH, C))
    nhi = jnp.maximum(nu[0, 0] - 2, 0).astype(_I32)
    return _predict(
        T, ux.reshape(N), iso.reshape(N),
        jnp.full((16,), xmin[0, 0], _F32),
        jnp.full((16,), xmax[0, 0], _F32),
        jnp.full((16,), nhi, _I32),
    )


# speedup vs baseline: 8.0159x; 1.5995x over previous
import functools

import jax
import jax.numpy as jnp
from jax import lax
from jax.experimental import pallas as pl
from jax.experimental.pallas import tpu as pltpu
from jax.experimental.pallas import tpu_sc as plsc

N = 4096
NT = 16384
C = 128
NCH = N // C
RB = 512
NRB = N // RB
NW = 32
QPW = NT // NW
CLIP_LO = -2.0
CLIP_HI = 2.0
_F32 = jnp.float32
_I32 = jnp.int32


def _bitonic_pairs(x, ypay, p):
    for k in range(12):
        asc = ((p >> (k + 1)) & 1) == 0
        for j in range(k, -1, -1):
            if j < 7:
                ax, sz, s = 1, C, 1 << j
            else:
                ax, sz, s = 0, NCH, 1 << (j - 7)
            is_low = ((p >> j) & 1) == 0
            pvx = jnp.where(is_low, pltpu.roll(x, sz - s, axis=ax),
                            pltpu.roll(x, s, axis=ax))
            pvy = jnp.where(is_low, pltpu.roll(ypay, sz - s, axis=ax),
                            pltpu.roll(ypay, s, axis=ax))
            want_min = is_low == asc
            take = (want_min & (pvx < x)) | (~want_min & (pvx > x))
            x = jnp.where(take, pvx, x)
            ypay = jnp.where(take, pvy, ypay)
    return x, ypay


def _fit_body(x2d_ref, y2d_ref,
              ux_ref, iso_ref, nu_ref, xmin_ref, xmax_ref,
              sx_ref, sy_ref, seg_ref,
              uyc_ref, uy2d_ref, s0_ref, sex_ref, scol_ref, carry_ref):
    lane = lax.broadcasted_iota(_I32, (1, C), 1)
    sub128 = lax.broadcasted_iota(_I32, (C, 1), 0)
    sub512 = lax.broadcasted_iota(_I32, (RB, 1), 0)
    sub32 = lax.broadcasted_iota(_I32, (NCH, 1), 0)
    lanei = lax.broadcasted_iota(_I32, (NCH, C), 1)
    subi = lax.broadcasted_iota(_I32, (NCH, C), 0)
    eye = sub128 == lane
    NEG = _F32(-jnp.inf)
    POS = _F32(jnp.inf)

    sx2d, sy2d = _bitonic_pairs(x2d_ref[...], y2d_ref[...],
                                subi * C + lanei)
    sx_ref[...] = sx2d
    sy_ref[...] = sy2d

    def prev_elem(v, fill):
        r1 = pltpu.roll(v, 1, axis=1)
        pv = jnp.where(lanei == 0, pltpu.roll(r1, 1, axis=0), r1)
        return jnp.where(subi * C + lanei == 0, fill, pv)

    def cumsum2d(v):
        s = 1
        while s < C:
            sh = pltpu.roll(v, s, axis=1)
            v = v + jnp.where(lanei >= s, sh, jnp.zeros((), v.dtype))
            s *= 2
        rt = v[:, C - 1:C]
        t = rt
        s = 1
        while s < NCH:
            sh = pltpu.roll(t, s, axis=0)
            t = t + jnp.where(sub32 >= s, sh, jnp.zeros((), v.dtype))
            s *= 2
        return v + (t - rt)

    flag = (subi * C + lanei == 0) | (sx2d != prev_elem(sx2d, NEG))
    flag_i = flag.astype(_I32)
    seg_ref[...] = cumsum2d(flag_i) - 1
    n_u = jnp.sum(jnp.sum(flag_i, axis=1, keepdims=True),
                  axis=0, keepdims=True)
    n_u_s = n_u[0, 0]
    band = (_I32(RB - 1) + (_I32(N) - n_u_s)) // _I32(C) + 1

    def s4(rb, _):
        ug = rb * RB + sub512
        c0 = rb * (RB // C)

        def inner(t, carry):
            cnt, sm, mx = carry
            c = c0 + t
            oh = seg_ref[pl.ds(c, 1), :] == ug
            syc = sy_ref[pl.ds(c, 1), :]
            sxc = sx_ref[pl.ds(c, 1), :]
            cnt = cnt + jnp.sum(jnp.where(oh, _F32(1.0), _F32(0.0)),
                                axis=1, keepdims=True)
            sm = sm + jnp.sum(jnp.where(oh, syc, _F32(0.0)),
                              axis=1, keepdims=True)
            mx = jnp.maximum(mx, jnp.max(jnp.where(oh, sxc, NEG),
                                         axis=1, keepdims=True))
            return cnt, sm, mx

        ntr = jnp.minimum(band, _I32(NCH) - c0)
        cnt, sm, mx = lax.fori_loop(
            0, ntr, inner,
            (jnp.zeros((RB, 1), _F32), jnp.zeros((RB, 1), _F32),
             jnp.full((RB, 1), NEG)))
        uyc_ref[pl.ds(rb * RB, RB), :] = sm / jnp.maximum(cnt, _F32(1.0))
        ux_ref[pl.ds(rb * RB, RB), :] = jnp.where(ug >= n_u, POS, mx)
        return 0
    lax.fori_loop(0, NRB, s4, 0, unroll=False)

    def t_uy(c, _):
        blk = uyc_ref[pl.ds(c * C, C), :]
        uy2d_ref[pl.ds(c, 1), :] = jnp.sum(
            jnp.where(eye, blk, _F32(0.0)), axis=0, keepdims=True)
        return 0
    lax.fori_loop(0, NCH, t_uy, 0, unroll=False)

    s_incl = cumsum2d(uy2d_ref[...])
    s0_ref[...] = s_incl
    sex_ref[...] = prev_elem(s_incl, _F32(0.0))

    def t_se(c, _):
        blk = sex_ref[pl.ds(c, 1), :]
        scol_ref[pl.ds(c * C, C), :] = jnp.sum(
            jnp.where(eye, blk, _F32(0.0)), axis=1, keepdims=True)
        return 0
    lax.fori_loop(0, NCH, t_se, 0, unroll=False)

    carry_ref[...] = jnp.full((N, 1), POS)

    def s5(t, _):
        cb = NCH - 1 - t
        kg = cb * C + lane
        sk = s0_ref[pl.ds(cb, 1), :]

        def inner(rb, isoacc):
            jb = rb * C + sub128
            sj = scol_ref[pl.ds(rb * C, C), :]
            d = kg - jb + 1
            m = (sk - sj) / jnp.maximum(d, 1).astype(_F32)
            m = jnp.where((d >= 1) & (kg < n_u), m, POS)
            s = 1
            while s < C:
                sh = pltpu.roll(m, C - s, axis=1)
                m = jnp.minimum(m, jnp.where(lane < C - s, sh, POS))
                s *= 2
            a = jnp.minimum(m, carry_ref[pl.ds(rb * C, C), :])
            carry_ref[pl.ds(rb * C, C), :] = a[:, 0:1]
            return jnp.maximum(isoacc,
                               jnp.max(jnp.where(jb <= kg, a, NEG),
                                       axis=0, keepdims=True))

        isoacc = lax.fori_loop(0, cb + 1, inner, jnp.full((1, C), NEG))
        iso_ref[pl.ds(cb, 1), :] = jnp.clip(isoacc, _F32(CLIP_LO),
                                            _F32(CLIP_HI))
        return 0
    lax.fori_loop(0, NCH, s5, 0, unroll=False)

    nu_ref[...] = n_u
    x2d = x2d_ref[...]
    xmin_ref[...] = jnp.min(jnp.min(x2d, axis=1, keepdims=True),
                            axis=0, keepdims=True)
    xmax_ref[...] = jnp.max(jnp.max(x2d, axis=1, keepdims=True),
                            axis=0, keepdims=True)


_FIT_OUT = [
    jax.ShapeDtypeStruct((N, 1), _F32),
    jax.ShapeDtypeStruct((NCH, C), _F32),
    jax.ShapeDtypeStruct((1, 1), _I32),
    jax.ShapeDtypeStruct((1, 1), _F32),
    jax.ShapeDtypeStruct((1, 1), _F32),
]

_FIT_SCRATCH = [
    pltpu.VMEM((NCH, C), _F32),
    pltpu.VMEM((NCH, C), _F32),
    pltpu.VMEM((NCH, C), _I32),
    pltpu.VMEM((N, 1), _F32),
    pltpu.VMEM((NCH, C), _F32),
    pltpu.VMEM((NCH, C), _F32),
    pltpu.VMEM((NCH, C), _F32),
    pltpu.VMEM((N, 1), _F32),
    pltpu.VMEM((N, 1), _F32),
]

_fit = pl.pallas_call(
    _fit_body,
    out_shape=_FIT_OUT,
    scratch_shapes=_FIT_SCRATCH,
)


def _predict(T, ux1, iso1, xmin16, xmax16, nhi16):
    mesh = plsc.VectorSubcoreMesh(core_axis_name="c", subcore_axis_name="s")

    @functools.partial(
        pl.kernel, mesh=mesh,
        out_type=jax.ShapeDtypeStruct((NT,), _F32),
        compiler_params=pltpu.CompilerParams(needs_layout_passes=False),
        scratch_types=[
            pltpu.VMEM((N,), _F32),
            pltpu.VMEM((N,), _F32),
            pltpu.VMEM((QPW,), _F32),
            pltpu.VMEM((QPW,), _F32),
            pltpu.VMEM((16,), _F32),
            pltpu.VMEM((16,), _F32),
            pltpu.VMEM((16,), _I32),
        ],
    )
    def k(t_hbm, ux_hbm, iso_hbm, xmin_hbm, xmax_hbm, nhi_hbm, out_hbm,
          ux_v, iso_v, t_v, o_v, xmin_v, xmax_v, nhi_v):
        wid = lax.axis_index("s") * 2 + lax.axis_index("c")
        base = wid * QPW
        pltpu.sync_copy(ux_hbm, ux_v)
        pltpu.sync_copy(iso_hbm, iso_v)
        pltpu.sync_copy(t_hbm.at[pl.ds(base, QPW)], t_v)
        pltpu.sync_copy(xmin_hbm, xmin_v)
        pltpu.sync_copy(xmax_hbm, xmax_v)
        pltpu.sync_copy(nhi_hbm, nhi_v)
        xmin = xmin_v[...]
        xmax = xmax_v[...]
        nhi = nhi_v[...]

        def body(g, acc):
            t = t_v[pl.ds(g * 16, 16)]
            tc = jnp.minimum(jnp.maximum(t, xmin), xmax)
            pos = jnp.zeros((16,), _I32)
            s = N // 2
            while s >= 1:
                cand = pos + s
                probe = plsc.load_gather(ux_v, [cand - 1])
                pos = jnp.where(probe <= tc, cand, pos)
                s //= 2
            idx = jnp.clip(pos - 1, 0, nhi)
            xb = plsc.load_gather(ux_v, [idx])
            xa = plsc.load_gather(ux_v, [idx + 1])
            yb = plsc.load_gather(iso_v, [idx])
            ya = plsc.load_gather(iso_v, [idx + 1])
            slope = (ya - yb) / (xa - xb)
            o_v[pl.ds(g * 16, 16)] = yb + slope * (tc - xb)
            return acc

        lax.fori_loop(0, QPW // 16, body, 0)
        pltpu.sync_copy(o_v, out_hbm.at[pl.ds(base, QPW)])

    return k(T, ux1, iso1, xmin16, xmax16, nhi16)


def kernel(X, y, T):
    ux, iso, nu, xmin, xmax = _fit(X.reshape(NCH, C), y.reshape(NCH, C))
    nhi = jnp.maximum(nu[0, 0] - 2, 0).astype(_I32)
    return _predict(
        T, ux.reshape(N), iso.reshape(N),
        jnp.full((16,), xmin[0, 0], _F32),
        jnp.full((16,), xmax[0, 0], _F32),
        jnp.full((16,), nhi, _I32),
    )
